# Initial kernel scaffold; baseline (speedup 1.0000x reference)
import jax
import jax.numpy as jnp
from jax.experimental import pallas as pl


def kernel(x, edge_index, W1, att_src1, att_dst1, bias1, W2, att_src2, att_dst2, bias2):
    raise NotImplementedError("write your pallas kernel here")


# speedup vs baseline: 41.2876x; 41.2876x over previous
import functools

import jax
import jax.numpy as jnp
from jax import lax
from jax.experimental import pallas as pl
from jax.experimental.pallas import tpu as pltpu
from jax.experimental.pallas import tpu_sc as plsc


def _dense_a_body(x_ref, waug_ref, wad_ref, aug_ref, ad_ref):
  x = x_ref[...]
  aug_ref[...] = jnp.dot(x, waug_ref[...], preferred_element_type=jnp.float32)
  ad_ref[...] = jnp.dot(x, wad_ref[...], preferred_element_type=jnp.float32)


def _dense_c_body(accp_ref, expand_ref, bias_ref, w2_ref, w2aug_ref, w2ad_ref,
                  aug2_ref, ad2_ref):
  acc = accp_ref[0] + accp_ref[1]
  num = acc[:, :128]
  den = acc[:, 128:144]
  denb = jnp.dot(den, expand_ref[...], preferred_element_type=jnp.float32)
  z = num / (denb + 1e-16) + bias_ref[...]
  h1a = jnp.where(z > 0, z, jnp.exp(z) - 1.0)
  h2 = jnp.dot(h1a, w2_ref[...], preferred_element_type=jnp.float32)
  aug2_ref[...] = jnp.dot(h2, w2aug_ref[...], preferred_element_type=jnp.float32)
  ad2_ref[...] = jnp.dot(h2, w2ad_ref[...], preferred_element_type=jnp.float32)


def _dense_e_body(accp_ref, avg_ref, bias_ref, out_ref):
  acc = accp_ref[0] + accp_ref[1]
  num = acc[:, :64]
  den = acc[:, 64:80]
  denb = jnp.dot(den, avg_ref[...], preferred_element_type=jnp.float32)
  out_ref[...] = num / (denb + 1e-16) + bias_ref[...]


def _row_grid_call(body, n, blk, in_shapes, out_shapes, blocked_in, blocked_out):
  grid = n // blk
  in_specs = []
  for shape, b in zip(in_shapes, blocked_in):
    if b:
      nlead = len(shape) - 2
      in_specs.append(pl.BlockSpec(
          shape[:nlead] + (blk, shape[-1]),
          lambda i, _n=nlead: (0,) * _n + (i, 0)))
    else:
      in_specs.append(pl.BlockSpec(shape, lambda i, _r=len(shape): (0,) * _r))
  out_specs = []
  for shape, b in zip(out_shapes, blocked_out):
    assert b
    out_specs.append(pl.BlockSpec((blk, shape[-1]), lambda i: (i, 0)))
  return pl.pallas_call(
      body,
      grid=(grid,),
      in_specs=in_specs,
      out_specs=out_specs[0] if len(out_specs) == 1 else out_specs,
      out_shape=[jax.ShapeDtypeStruct(s, jnp.float32) for s in out_shapes]
      if len(out_shapes) > 1 else jax.ShapeDtypeStruct(out_shapes[0], jnp.float32),
  )


def _make_edge_aggregator(n, e, feat, logit_replicated):
  roww = feat + 16
  info = plsc.get_sparse_core_info()
  nc, ns = info.num_cores, info.num_subcores
  nw = nc * ns
  ew = e // nw
  C = 80
  assert ew % C == 0 and e % nw == 0
  nchunks = ew // C
  rps = 8 * (-(-n // (ns * 8)))
  nz = -(-rps // C)

  mesh = plsc.VectorSubcoreMesh(core_axis_name="c", subcore_axis_name="s")

  @functools.partial(
      pl.kernel,
      mesh=mesh,
      compiler_params=pltpu.CompilerParams(use_tc_tiling_on_sc=False),
      out_type=jax.ShapeDtypeStruct((nc, n, roww), jnp.float32),
      scratch_types=[
          pltpu.VMEM((C,), jnp.int32),
          pltpu.VMEM((C,), jnp.int32),
          pltpu.VMEM((C, roww), jnp.float32),
          pltpu.VMEM((C, 16), jnp.float32),
          pltpu.VMEM((C, roww), jnp.float32),
          pltpu.VMEM((16,), jnp.float32),
          pltpu.VMEM_SHARED((n, roww), jnp.float32),
          pltpu.SemaphoreType.DMA,
      ],
  )
  def agg(aug_hbm, ad_hbm, src_hbm, dst_hbm, out_hbm,
          src_v, dst_v, rows_v, adrows_v, vals_v, pbuf_v, acc_sh, sem):
    cid = lax.axis_index("c")
    sid = lax.axis_index("s")
    wid = sid * nc + cid
    zero16 = jnp.zeros((16,), jnp.float32)

    def zrow(i, _):
      for g in range(roww // 16):
        vals_v[i, pl.ds(16 * g, 16)] = zero16
      return 0
    lax.fori_loop(0, C, zrow, 0)

    r0 = jnp.minimum(sid * rps, n - rps)
    def zchunk(i, _):
      z0 = r0 + jnp.minimum(i * C, rps - C)
      pltpu.sync_copy(vals_v, acc_sh.at[pl.ds(z0, C)])
      return 0
    lax.fori_loop(0, nz, zchunk, 0)
    plsc.subcore_barrier()

    base = wid * ew

    def chunk(i, _):
      e0 = base + i * C
      pltpu.sync_copy(src_hbm.at[pl.ds(e0, C)], src_v)
      pltpu.sync_copy(dst_hbm.at[pl.ds(e0, C)], dst_v)
      pltpu.async_copy(aug_hbm.at[src_v], rows_v, sem).wait()
      pltpu.async_copy(ad_hbm.at[dst_v], adrows_v, sem).wait()

      def edge(k, _):
        a = rows_v[k, pl.ds(feat, 16)] + adrows_v[k]
        a = jnp.where(a > 0, a, 0.2 * a)
        p = jnp.exp(a)
        vals_v[k, pl.ds(feat, 16)] = p
        if logit_replicated:
          for g in range(feat // 16):
            vals_v[k, pl.ds(16 * g, 16)] = rows_v[k, pl.ds(16 * g, 16)] * p
        else:
          for h in range(feat // 16):
            vals_v[k, pl.ds(16 * h, 16)] = rows_v[k, pl.ds(16 * h, 16)] * p[h]
        return 0
      lax.fori_loop(0, C, edge, 0)

      pltpu.sync_copy(vals_v, acc_sh.at[dst_v], add=True)
      return 0
    lax.fori_loop(0, nchunks, chunk, 0)
    plsc.subcore_barrier()

    pltpu.sync_copy(acc_sh.at[pl.ds(r0, rps)],
                    out_hbm.at[cid, pl.ds(r0, rps)])

  return agg


def kernel(x, edge_index, W1, att_src1, att_dst1, bias1,
           W2, att_src2, att_dst2, bias2):
  n, in_dim = x.shape
  e = edge_index.shape[1]
  heads, hid = att_src1.shape
  out_dim = W2.shape[1]
  f1 = heads * hid
  blk = 1000

  src = edge_index[0]
  dst = edge_index[1]

  head_of_col = jnp.arange(f1) // hid
  onehot = jnp.eye(heads, dtype=jnp.float32)[head_of_col]
  As = onehot * att_src1.reshape(f1)[:, None]
  Ad = onehot * att_dst1.reshape(f1)[:, None]
  pad8 = jnp.zeros((f1, 16 - heads), jnp.float32)
  As16 = jnp.concatenate([As, pad8], axis=1)
  Ad16 = jnp.concatenate([Ad, pad8], axis=1)
  Waug = jnp.concatenate([W1, jnp.dot(W1, As16)], axis=1)
  Wad = jnp.dot(W1, Ad16)
  expand = jnp.concatenate(
      [onehot.T, jnp.zeros((16 - heads, f1), jnp.float32)], axis=0)
  ones16 = jnp.ones((1, 16), jnp.float32)
  AS2rep = att_src2.reshape(out_dim, 1) @ ones16
  AD2rep = att_dst2.reshape(out_dim, 1) @ ones16
  W2aug = jnp.concatenate([jnp.eye(out_dim, dtype=jnp.float32), AS2rep],
                          axis=1)
  avg = jnp.full((16, out_dim), 1.0 / 16.0, jnp.float32)
  bias1r = bias1.reshape(1, f1)
  bias2r = bias2.reshape(1, out_dim)

  aug1, ad1 = _row_grid_call(
      _dense_a_body, n, blk,
      [(n, in_dim), (in_dim, f1 + 16), (in_dim, 16)],
      [(n, f1 + 16), (n, 16)],
      [True, False, False], [True, True])(x, Waug, Wad)

  accp1 = _make_edge_aggregator(n, e, f1, logit_replicated=False)(
      aug1, ad1, src, dst)

  aug2, ad2 = _row_grid_call(
      _dense_c_body, n, blk,
      [(2, n, f1 + 16), (16, f1), (1, f1), (f1, out_dim), (out_dim, out_dim + 16),
       (out_dim, 16)],
      [(n, out_dim + 16), (n, 16)],
      [True, False, False, False, False, False], [True, True])(
          accp1, expand, bias1r, W2, W2aug, AD2rep)

  accp2 = _make_edge_aggregator(n, e, out_dim, logit_replicated=True)(
      aug2, ad2, src, dst)

  out = _row_grid_call(
      _dense_e_body, n, blk,
      [(2, n, out_dim + 16), (16, out_dim), (1, out_dim)],
      [(n, out_dim)],
      [True, False, False], [True])(accp2, avg, bias2r)
  return out


# speedup vs baseline: 61.9922x; 1.5015x over previous
import functools

import jax
import jax.numpy as jnp
from jax import lax
from jax.experimental import pallas as pl
from jax.experimental.pallas import tpu as pltpu
from jax.experimental.pallas import tpu_sc as plsc


def _dense_a_body(x_ref, waug_ref, wad_ref, aug_ref, ad_ref):
  x = x_ref[...]
  aug_ref[...] = jnp.dot(x, waug_ref[...], preferred_element_type=jnp.float32)
  ad_ref[...] = jnp.dot(x, wad_ref[...], preferred_element_type=jnp.float32)


def _dense_c_body(accp_ref, expand_ref, bias_ref, w2_ref, w2aug_ref, w2ad_ref,
                  aug2_ref, ad2_ref):
  acc = accp_ref[0] + accp_ref[1]
  num = acc[:, :128]
  den = acc[:, 128:144]
  denb = jnp.dot(den, expand_ref[...], preferred_element_type=jnp.float32)
  z = num / (denb + 1e-16) + bias_ref[...]
  h1a = jnp.where(z > 0, z, jnp.exp(z) - 1.0)
  h2 = jnp.dot(h1a, w2_ref[...], preferred_element_type=jnp.float32)
  aug2_ref[...] = jnp.dot(h2, w2aug_ref[...], preferred_element_type=jnp.float32)
  ad2_ref[...] = jnp.dot(h2, w2ad_ref[...], preferred_element_type=jnp.float32)


def _dense_e_body(accp_ref, avg_ref, bias_ref, out_ref):
  acc = accp_ref[0] + accp_ref[1]
  num = acc[:, :64]
  den = acc[:, 64:80]
  denb = jnp.dot(den, avg_ref[...], preferred_element_type=jnp.float32)
  out_ref[...] = num / (denb + 1e-16) + bias_ref[...]


def _row_grid_call(body, n, blk, in_shapes, out_shapes, blocked_in, blocked_out):
  grid = n // blk
  in_specs = []
  for shape, b in zip(in_shapes, blocked_in):
    if b:
      nlead = len(shape) - 2
      in_specs.append(pl.BlockSpec(
          shape[:nlead] + (blk, shape[-1]),
          lambda i, _n=nlead: (0,) * _n + (i, 0)))
    else:
      in_specs.append(pl.BlockSpec(shape, lambda i, _r=len(shape): (0,) * _r))
  out_specs = []
  for shape, b in zip(out_shapes, blocked_out):
    assert b
    out_specs.append(pl.BlockSpec((blk, shape[-1]), lambda i: (i, 0)))
  return pl.pallas_call(
      body,
      grid=(grid,),
      in_specs=in_specs,
      out_specs=out_specs[0] if len(out_specs) == 1 else out_specs,
      out_shape=[jax.ShapeDtypeStruct(s, jnp.float32) for s in out_shapes]
      if len(out_shapes) > 1 else jax.ShapeDtypeStruct(out_shapes[0], jnp.float32),
  )


def _make_edge_aggregator(n, e, feat, logit_replicated):
  roww = feat + 16
  info = plsc.get_sparse_core_info()
  nc, ns = info.num_cores, info.num_subcores
  nw = nc * ns
  ew = e // nw
  C = 80
  assert ew % C == 0 and e % nw == 0
  nchunks = ew // C
  rps = 8 * (-(-n // (ns * 8)))
  nz = -(-rps // C)

  mesh = plsc.VectorSubcoreMesh(core_axis_name="c", subcore_axis_name="s")

  @functools.partial(
      pl.kernel,
      mesh=mesh,
      compiler_params=pltpu.CompilerParams(use_tc_tiling_on_sc=False),
      out_type=jax.ShapeDtypeStruct((nc, n, roww), jnp.float32),
      scratch_types=[
          pltpu.VMEM((C,), jnp.int32),
          pltpu.VMEM((C,), jnp.int32),
          pltpu.VMEM((C,), jnp.int32),
          pltpu.VMEM((C,), jnp.int32),
          pltpu.VMEM((C, roww), jnp.float32),
          pltpu.VMEM((C, roww), jnp.float32),
          pltpu.VMEM((C, 16), jnp.float32),
          pltpu.VMEM((C, 16), jnp.float32),
          pltpu.VMEM_SHARED((n, roww), jnp.float32),
          pltpu.SemaphoreType.DMA,
          pltpu.SemaphoreType.DMA,
          pltpu.SemaphoreType.DMA,
          pltpu.SemaphoreType.DMA,
      ],
  )
  def agg(aug_hbm, ad_hbm, src_hbm, dst_hbm, out_hbm,
          src_a, src_b, dst_a, dst_b, rows_a, rows_b, ad_a, ad_b,
          acc_sh, gsem_a, gsem_b, ssem_a, ssem_b):
    cid = lax.axis_index("c")
    sid = lax.axis_index("s")
    wid = sid * nc + cid
    zero16 = jnp.zeros((16,), jnp.float32)
    bufs = ((src_a, dst_a, rows_a, ad_a, gsem_a, ssem_a),
            (src_b, dst_b, rows_b, ad_b, gsem_b, ssem_b))

    def zrow(i, _):
      for g in range(roww // 16):
        rows_a[i, pl.ds(16 * g, 16)] = zero16
      return 0
    lax.fori_loop(0, C, zrow, 0)

    r0 = jnp.minimum(sid * rps, n - rps)
    def zchunk(i, _):
      z0 = r0 + jnp.minimum(i * C, rps - C)
      pltpu.sync_copy(rows_a, acc_sh.at[pl.ds(z0, C)])
      return 0
    lax.fori_loop(0, nz, zchunk, 0)
    plsc.subcore_barrier()

    base = wid * ew

    def prefetch(e0, buf):
      src_v, dst_v, rows_v, adrows_v, gsem, _ = buf
      pltpu.sync_copy(src_hbm.at[pl.ds(e0, C)], src_v)
      pltpu.sync_copy(dst_hbm.at[pl.ds(e0, C)], dst_v)
      pltpu.async_copy(aug_hbm.at[src_v], rows_v, gsem)
      pltpu.async_copy(ad_hbm.at[dst_v], adrows_v, gsem)

    def wait_gathers(buf):
      src_v, dst_v, rows_v, adrows_v, gsem, _ = buf
      pltpu.make_async_copy(aug_hbm.at[src_v], rows_v, gsem).wait()
      pltpu.make_async_copy(ad_hbm.at[dst_v], adrows_v, gsem).wait()

    def wait_scatter(buf):
      _, dst_v, rows_v, _, _, ssem = buf
      pltpu.make_async_copy(rows_v, acc_sh.at[dst_v], ssem).wait()

    def compute_and_scatter(buf):
      _, dst_v, rows_v, adrows_v, _, ssem = buf

      def edge(k, _):
        a = rows_v[k, pl.ds(feat, 16)] + adrows_v[k]
        a = jnp.where(a > 0, a, 0.2 * a)
        p = jnp.exp(a)
        if logit_replicated:
          for g in range(feat // 16):
            rows_v[k, pl.ds(16 * g, 16)] = rows_v[k, pl.ds(16 * g, 16)] * p
        else:
          for h in range(feat // 16):
            rows_v[k, pl.ds(16 * h, 16)] = rows_v[k, pl.ds(16 * h, 16)] * p[h]
        rows_v[k, pl.ds(feat, 16)] = p
        return 0
      lax.fori_loop(0, C, edge, 0)
      pltpu.async_copy(rows_v, acc_sh.at[dst_v], ssem, add=True)

    prefetch(base, bufs[0])

    def chunk(i, _):
      for par in (0, 1):
        @pl.when((i & 1) == par)
        def _():
          cur, nxt = bufs[par], bufs[1 - par]

          @pl.when(i >= 1)
          def _():
            wait_scatter(nxt)

          @pl.when(i + 1 < nchunks)
          def _():
            prefetch(base + (i + 1) * C, nxt)

          wait_gathers(cur)
          compute_and_scatter(cur)
      return 0
    lax.fori_loop(0, nchunks, chunk, 0)
    wait_scatter(bufs[(nchunks - 1) & 1])
    plsc.subcore_barrier()

    pltpu.sync_copy(acc_sh.at[pl.ds(r0, rps)],
                    out_hbm.at[cid, pl.ds(r0, rps)])

  return agg


def kernel(x, edge_index, W1, att_src1, att_dst1, bias1,
           W2, att_src2, att_dst2, bias2):
  n, in_dim = x.shape
  e = edge_index.shape[1]
  heads, hid = att_src1.shape
  out_dim = W2.shape[1]
  f1 = heads * hid
  blk = 1000

  src = edge_index[0]
  dst = edge_index[1]

  head_of_col = jnp.arange(f1) // hid
  onehot = jnp.eye(heads, dtype=jnp.float32)[head_of_col]
  As = onehot * att_src1.reshape(f1)[:, None]
  Ad = onehot * att_dst1.reshape(f1)[:, None]
  pad8 = jnp.zeros((f1, 16 - heads), jnp.float32)
  As16 = jnp.concatenate([As, pad8], axis=1)
  Ad16 = jnp.concatenate([Ad, pad8], axis=1)
  Waug = jnp.concatenate([W1, jnp.dot(W1, As16)], axis=1)
  Wad = jnp.dot(W1, Ad16)
  expand = jnp.concatenate(
      [onehot.T, jnp.zeros((16 - heads, f1), jnp.float32)], axis=0)
  ones16 = jnp.ones((1, 16), jnp.float32)
  AS2rep = att_src2.reshape(out_dim, 1) @ ones16
  AD2rep = att_dst2.reshape(out_dim, 1) @ ones16
  W2aug = jnp.concatenate([jnp.eye(out_dim, dtype=jnp.float32), AS2rep],
                          axis=1)
  avg = jnp.full((16, out_dim), 1.0 / 16.0, jnp.float32)
  bias1r = bias1.reshape(1, f1)
  bias2r = bias2.reshape(1, out_dim)

  aug1, ad1 = _row_grid_call(
      _dense_a_body, n, blk,
      [(n, in_dim), (in_dim, f1 + 16), (in_dim, 16)],
      [(n, f1 + 16), (n, 16)],
      [True, False, False], [True, True])(x, Waug, Wad)

  accp1 = _make_edge_aggregator(n, e, f1, logit_replicated=False)(
      aug1, ad1, src, dst)

  aug2, ad2 = _row_grid_call(
      _dense_c_body, n, blk,
      [(2, n, f1 + 16), (16, f1), (1, f1), (f1, out_dim), (out_dim, out_dim + 16),
       (out_dim, 16)],
      [(n, out_dim + 16), (n, 16)],
      [True, False, False, False, False, False], [True, True])(
          accp1, expand, bias1r, W2, W2aug, AD2rep)

  accp2 = _make_edge_aggregator(n, e, out_dim, logit_replicated=True)(
      aug2, ad2, src, dst)

  out = _row_grid_call(
      _dense_e_body, n, blk,
      [(2, n, out_dim + 16), (16, out_dim), (1, out_dim)],
      [(n, out_dim)],
      [True, False, False], [True])(accp2, avg, bias2r)
  return out


# speedup vs baseline: 70.6844x; 1.1402x over previous
import functools

import jax
import jax.numpy as jnp
from jax import lax
from jax.experimental import pallas as pl
from jax.experimental.pallas import tpu as pltpu
from jax.experimental.pallas import tpu_sc as plsc


def _dense_a_body(x_ref, waug_ref, wad_ref, aug_ref, ad_ref):
  x = x_ref[...]
  aug_ref[...] = jnp.dot(x, waug_ref[...], preferred_element_type=jnp.float32)
  ad_ref[...] = jnp.dot(x, wad_ref[...], preferred_element_type=jnp.float32)


def _dense_c_body(accp_ref, expand_ref, bias_ref, w2_ref, w2aug_ref, w2ad_ref,
                  aug2_ref, ad2_ref):
  acc = accp_ref[0] + accp_ref[1]
  num = acc[:, :128]
  den = acc[:, 128:144]
  denb = jnp.dot(den, expand_ref[...], preferred_element_type=jnp.float32)
  z = num / (denb + 1e-16) + bias_ref[...]
  h1a = jnp.where(z > 0, z, jnp.exp(z) - 1.0)
  h2 = jnp.dot(h1a, w2_ref[...], preferred_element_type=jnp.float32)
  aug2_ref[...] = jnp.dot(h2, w2aug_ref[...], preferred_element_type=jnp.float32)
  ad2_ref[...] = jnp.dot(h2, w2ad_ref[...], preferred_element_type=jnp.float32)


def _dense_e_body(accp_ref, avg_ref, bias_ref, out_ref):
  acc = accp_ref[0] + accp_ref[1]
  num = acc[:, :64]
  den = acc[:, 64:80]
  denb = jnp.dot(den, avg_ref[...], preferred_element_type=jnp.float32)
  out_ref[...] = num / (denb + 1e-16) + bias_ref[...]


def _row_grid_call(body, n, blk, in_shapes, out_shapes, blocked_in, blocked_out):
  grid = n // blk
  in_specs = []
  for shape, b in zip(in_shapes, blocked_in):
    if b:
      nlead = len(shape) - 2
      in_specs.append(pl.BlockSpec(
          shape[:nlead] + (blk, shape[-1]),
          lambda i, _n=nlead: (0,) * _n + (i, 0)))
    else:
      in_specs.append(pl.BlockSpec(shape, lambda i, _r=len(shape): (0,) * _r))
  out_specs = []
  for shape, b in zip(out_shapes, blocked_out):
    assert b
    out_specs.append(pl.BlockSpec((blk, shape[-1]), lambda i: (i, 0)))
  return pl.pallas_call(
      body,
      grid=(grid,),
      in_specs=in_specs,
      out_specs=out_specs[0] if len(out_specs) == 1 else out_specs,
      out_shape=[jax.ShapeDtypeStruct(s, jnp.float32) for s in out_shapes]
      if len(out_shapes) > 1 else jax.ShapeDtypeStruct(out_shapes[0], jnp.float32),
  )


def _make_edge_aggregator(n, e, feat, logit_replicated):
  roww = feat + 16
  C = 100
  info = plsc.get_sparse_core_info()
  nc, ns = info.num_cores, info.num_subcores
  nw = nc * ns
  assert e % (C * nw) == 0
  nchunks = e // (C * nw)
  rps = 8 * (-(-n // (ns * 8)))
  ZR = 96
  nz = -(-rps // ZR)

  mesh = plsc.VectorSubcoreMesh(core_axis_name="c", subcore_axis_name="s")

  @functools.partial(
      pl.kernel,
      mesh=mesh,
      compiler_params=pltpu.CompilerParams(use_tc_tiling_on_sc=False),
      out_type=jax.ShapeDtypeStruct((nc, n, roww), jnp.float32),
      scratch_types=[
          pltpu.VMEM((2, C), jnp.int32),
          pltpu.VMEM((2, C), jnp.int32),
          pltpu.VMEM((C, roww), jnp.float32),
          pltpu.VMEM((C, roww), jnp.float32),
          pltpu.VMEM((C, 16), jnp.float32),
          pltpu.VMEM((C, 16), jnp.float32),
          pltpu.VMEM_SHARED((n, roww), jnp.float32),
          pltpu.SemaphoreType.DMA,
          pltpu.SemaphoreType.DMA,
          pltpu.SemaphoreType.DMA,
          pltpu.SemaphoreType.DMA,
      ],
  )
  def agg(aug_hbm, ad_hbm, idx_hbm, out_hbm,
          idx_a, idx_b, rows_a, rows_b, ad_a, ad_b,
          acc_sh, gsem_a, gsem_b, ssem_a, ssem_b):
    cid = lax.axis_index("c")
    sid = lax.axis_index("s")
    wid = sid * nc + cid
    zero16 = jnp.zeros((16,), jnp.float32)
    bufs = ((idx_a, rows_a, ad_a, gsem_a, ssem_a),
            (idx_b, rows_b, ad_b, gsem_b, ssem_b))

    def zrow(i, _):
      for g in range(roww // 16):
        rows_a[i, pl.ds(16 * g, 16)] = zero16
      return 0
    lax.fori_loop(0, C, zrow, 0)

    r0 = jnp.minimum(sid * rps, n - rps)
    def zchunk(i, _):
      z0 = r0 + jnp.minimum(i * ZR, rps - ZR)
      pltpu.sync_copy(rows_a.at[pl.ds(0, ZR)], acc_sh.at[pl.ds(z0, ZR)])
      return 0
    lax.fori_loop(0, nz, zchunk, 0)
    plsc.subcore_barrier()

    base = wid * nchunks

    def prefetch(g, buf):
      idx_v, rows_v, adrows_v, gsem, _ = buf
      pltpu.sync_copy(idx_hbm.at[g], idx_v)
      pltpu.async_copy(aug_hbm.at[idx_v.at[0]], rows_v, gsem)
      pltpu.async_copy(ad_hbm.at[idx_v.at[1]], adrows_v, gsem)

    def wait_gathers(buf):
      idx_v, rows_v, adrows_v, gsem, _ = buf
      pltpu.make_async_copy(aug_hbm.at[idx_v.at[0]], rows_v, gsem).wait()
      pltpu.make_async_copy(ad_hbm.at[idx_v.at[1]], adrows_v, gsem).wait()

    def wait_scatter(buf):
      idx_v, rows_v, _, _, ssem = buf
      pltpu.make_async_copy(rows_v, acc_sh.at[idx_v.at[1]], ssem).wait()

    def compute_and_scatter(buf):
      idx_v, rows_v, adrows_v, _, ssem = buf

      def edge(k, _):
        a = rows_v[k, pl.ds(feat, 16)] + adrows_v[k]
        a = jnp.where(a > 0, a, 0.2 * a)
        p = jnp.exp(a)
        if logit_replicated:
          for g in range(feat // 16):
            rows_v[k, pl.ds(16 * g, 16)] = rows_v[k, pl.ds(16 * g, 16)] * p
        else:
          for h in range(feat // 16):
            rows_v[k, pl.ds(16 * h, 16)] = rows_v[k, pl.ds(16 * h, 16)] * p[h]
        rows_v[k, pl.ds(feat, 16)] = p
        return 0
      lax.fori_loop(0, C, edge, 0)
      pltpu.async_copy(rows_v, acc_sh.at[idx_v.at[1]], ssem, add=True)

    prefetch(base, bufs[0])

    def chunk(i, _):
      for par in (0, 1):
        @pl.when((i & 1) == par)
        def _():
          cur, nxt = bufs[par], bufs[1 - par]

          @pl.when(i >= 1)
          def _():
            wait_scatter(nxt)

          @pl.when(i + 1 < nchunks)
          def _():
            prefetch(base + i + 1, nxt)

          wait_gathers(cur)
          compute_and_scatter(cur)
      return 0
    lax.fori_loop(0, nchunks, chunk, 0)
    wait_scatter(bufs[(nchunks - 1) & 1])
    plsc.subcore_barrier()

    pltpu.sync_copy(acc_sh.at[pl.ds(r0, rps)],
                    out_hbm.at[cid, pl.ds(r0, rps)])

  return agg


def kernel(x, edge_index, W1, att_src1, att_dst1, bias1,
           W2, att_src2, att_dst2, bias2):
  n, in_dim = x.shape
  e = edge_index.shape[1]
  heads, hid = att_src1.shape
  out_dim = W2.shape[1]
  f1 = heads * hid
  blk = 1000

  idx3d = jnp.stack([edge_index[0].reshape(-1, 100),
                     edge_index[1].reshape(-1, 100)], axis=1)

  head_of_col = jnp.arange(f1) // hid
  onehot = jnp.eye(heads, dtype=jnp.float32)[head_of_col]
  As = onehot * att_src1.reshape(f1)[:, None]
  Ad = onehot * att_dst1.reshape(f1)[:, None]
  pad8 = jnp.zeros((f1, 16 - heads), jnp.float32)
  As16 = jnp.concatenate([As, pad8], axis=1)
  Ad16 = jnp.concatenate([Ad, pad8], axis=1)
  Waug = jnp.concatenate([W1, jnp.dot(W1, As16)], axis=1)
  Wad = jnp.dot(W1, Ad16)
  expand = jnp.concatenate(
      [onehot.T, jnp.zeros((16 - heads, f1), jnp.float32)], axis=0)
  ones16 = jnp.ones((1, 16), jnp.float32)
  AS2rep = att_src2.reshape(out_dim, 1) @ ones16
  AD2rep = att_dst2.reshape(out_dim, 1) @ ones16
  W2aug = jnp.concatenate([jnp.eye(out_dim, dtype=jnp.float32), AS2rep],
                          axis=1)
  avg = jnp.full((16, out_dim), 1.0 / 16.0, jnp.float32)
  bias1r = bias1.reshape(1, f1)
  bias2r = bias2.reshape(1, out_dim)

  aug1, ad1 = _row_grid_call(
      _dense_a_body, n, blk,
      [(n, in_dim), (in_dim, f1 + 16), (in_dim, 16)],
      [(n, f1 + 16), (n, 16)],
      [True, False, False], [True, True])(x, Waug, Wad)

  accp1 = _make_edge_aggregator(n, e, f1, logit_replicated=False)(
      aug1, ad1, idx3d)

  aug2, ad2 = _row_grid_call(
      _dense_c_body, n, blk,
      [(2, n, f1 + 16), (16, f1), (1, f1), (f1, out_dim), (out_dim, out_dim + 16),
       (out_dim, 16)],
      [(n, out_dim + 16), (n, 16)],
      [True, False, False, False, False, False], [True, True])(
          accp1, expand, bias1r, W2, W2aug, AD2rep)

  accp2 = _make_edge_aggregator(n, e, out_dim, logit_replicated=True)(
      aug2, ad2, idx3d)

  out = _row_grid_call(
      _dense_e_body, n, blk,
      [(2, n, out_dim + 16), (16, out_dim), (1, out_dim)],
      [(n, out_dim)],
      [True, False, False], [True])(accp2, avg, bias2r)
  return out


# speedup vs baseline: 125.3628x; 1.7736x over previous
import functools

import jax
import jax.numpy as jnp
from jax import lax
from jax.experimental import pallas as pl
from jax.experimental.pallas import tpu as pltpu
from jax.experimental.pallas import tpu_sc as plsc


def _dense_a_body(x_ref, waug_ref, wad_ref, aug_ref, ad_ref):
  x = x_ref[...]
  aug_ref[...] = jnp.dot(x, waug_ref[...], preferred_element_type=jnp.float32)
  ad_ref[...] = jnp.dot(x, wad_ref[...], preferred_element_type=jnp.float32)


def _dense_c_body(accp_ref, expand_ref, bias_ref, w2_ref, w2aug_ref, w2ad_ref,
                  aug2_ref, ad2_ref):
  acc = accp_ref[0] + accp_ref[1]
  num = acc[:, :128]
  den = acc[:, 128:144]
  denb = jnp.dot(den, expand_ref[...], preferred_element_type=jnp.float32)
  z = num / (denb + 1e-16) + bias_ref[...]
  h1a = jnp.where(z > 0, z, jnp.exp(z) - 1.0)
  h2 = jnp.dot(h1a, w2_ref[...], preferred_element_type=jnp.float32)
  aug2_ref[...] = jnp.dot(h2, w2aug_ref[...], preferred_element_type=jnp.float32)
  ad2_ref[...] = jnp.dot(h2, w2ad_ref[...], preferred_element_type=jnp.float32)


def _dense_e_body(accp_ref, avg_ref, bias_ref, out_ref):
  acc = accp_ref[0] + accp_ref[1]
  num = acc[:, :64]
  den = acc[:, 64:80]
  denb = jnp.dot(den, avg_ref[...], preferred_element_type=jnp.float32)
  out_ref[...] = num / (denb + 1e-16) + bias_ref[...]


def _row_grid_call(body, n, blk, in_shapes, out_shapes, blocked_in, blocked_out):
  grid = n // blk
  in_specs = []
  for shape, b in zip(in_shapes, blocked_in):
    if b:
      nlead = len(shape) - 2
      in_specs.append(pl.BlockSpec(
          shape[:nlead] + (blk, shape[-1]),
          lambda i, _n=nlead: (0,) * _n + (i, 0)))
    else:
      in_specs.append(pl.BlockSpec(shape, lambda i, _r=len(shape): (0,) * _r))
  out_specs = []
  for shape, b in zip(out_shapes, blocked_out):
    assert b
    out_specs.append(pl.BlockSpec((blk, shape[-1]), lambda i: (i, 0)))
  return pl.pallas_call(
      body,
      grid=(grid,),
      in_specs=in_specs,
      out_specs=out_specs[0] if len(out_specs) == 1 else out_specs,
      out_shape=[jax.ShapeDtypeStruct(s, jnp.float32) for s in out_shapes]
      if len(out_shapes) > 1 else jax.ShapeDtypeStruct(out_shapes[0], jnp.float32),
  )


def _make_edge_aggregator(n, e, feat, logit_replicated):
  roww = feat + 16
  C = 100
  info = plsc.get_sparse_core_info()
  nc, ns = info.num_cores, info.num_subcores
  nw = nc * ns
  assert e % (C * nw) == 0
  nchunks = e // (C * nw)
  rps = 8 * (-(-n // (ns * 8)))
  ZR = 96
  nz = -(-rps // ZR)

  mesh = plsc.VectorSubcoreMesh(core_axis_name="c", subcore_axis_name="s")

  @functools.partial(
      pl.kernel,
      mesh=mesh,
      compiler_params=pltpu.CompilerParams(use_tc_tiling_on_sc=False),
      out_type=jax.ShapeDtypeStruct((nc, n, roww), jnp.float32),
      scratch_types=[
          pltpu.VMEM((2, C), jnp.int32),
          pltpu.VMEM((2, C), jnp.int32),
          pltpu.VMEM((C, roww), jnp.float32),
          pltpu.VMEM((C, roww), jnp.float32),
          pltpu.VMEM((C, 16), jnp.float32),
          pltpu.VMEM((C, 16), jnp.float32),
          pltpu.VMEM_SHARED((n, roww), jnp.float32),
          pltpu.SemaphoreType.DMA,
          pltpu.SemaphoreType.DMA,
          pltpu.SemaphoreType.DMA,
          pltpu.SemaphoreType.DMA,
      ],
  )
  def agg(aug_hbm, ad_hbm, idx_hbm, out_hbm,
          idx_a, idx_b, rows_a, rows_b, ad_a, ad_b,
          acc_sh, gsem_a, gsem_b, ssem_a, ssem_b):
    cid = lax.axis_index("c")
    sid = lax.axis_index("s")
    wid = sid * nc + cid
    zero16 = jnp.zeros((16,), jnp.float32)
    bufs = ((idx_a, rows_a, ad_a, gsem_a, ssem_a),
            (idx_b, rows_b, ad_b, gsem_b, ssem_b))

    def zrow(i, _):
      for g in range(roww // 16):
        rows_a[i, pl.ds(16 * g, 16)] = zero16
      return 0
    lax.fori_loop(0, C, zrow, 0)

    r0 = jnp.minimum(sid * rps, n - rps)
    def zchunk(i, _):
      z0 = r0 + jnp.minimum(i * ZR, rps - ZR)
      pltpu.sync_copy(rows_a.at[pl.ds(0, ZR)], acc_sh.at[pl.ds(z0, ZR)])
      return 0
    lax.fori_loop(0, nz, zchunk, 0)
    plsc.subcore_barrier()

    base = wid * nchunks

    def prefetch(g, buf):
      idx_v, rows_v, adrows_v, gsem, _ = buf
      pltpu.sync_copy(idx_hbm.at[g], idx_v)
      pltpu.async_copy(aug_hbm.at[idx_v.at[0]], rows_v, gsem)
      pltpu.async_copy(ad_hbm.at[idx_v.at[1]], adrows_v, gsem)

    def wait_gathers(buf):
      idx_v, rows_v, adrows_v, gsem, _ = buf
      pltpu.make_async_copy(aug_hbm.at[idx_v.at[0]], rows_v, gsem).wait()
      pltpu.make_async_copy(ad_hbm.at[idx_v.at[1]], adrows_v, gsem).wait()

    def wait_scatter(buf):
      idx_v, rows_v, _, _, ssem = buf
      pltpu.make_async_copy(rows_v, acc_sh.at[idx_v.at[1]], ssem).wait()

    def compute_and_scatter(buf):
      idx_v, rows_v, adrows_v, _, ssem = buf

      @functools.partial(plsc.parallel_loop, 0, C, unroll=4)
      def edge(k):
        a = rows_v[k, pl.ds(feat, 16)] + adrows_v[k]
        a = jnp.where(a > 0, a, 0.2 * a)
        p = jnp.exp(a)
        if logit_replicated:
          for g in range(feat // 16):
            rows_v[k, pl.ds(16 * g, 16)] = rows_v[k, pl.ds(16 * g, 16)] * p
        else:
          for h in range(feat // 16):
            rows_v[k, pl.ds(16 * h, 16)] = rows_v[k, pl.ds(16 * h, 16)] * p[h]
        rows_v[k, pl.ds(feat, 16)] = p
      pltpu.async_copy(rows_v, acc_sh.at[idx_v.at[1]], ssem, add=True)

    prefetch(base, bufs[0])

    def chunk(i, _):
      for par in (0, 1):
        @pl.when((i & 1) == par)
        def _():
          cur, nxt = bufs[par], bufs[1 - par]

          @pl.when(i >= 1)
          def _():
            wait_scatter(nxt)

          @pl.when(i + 1 < nchunks)
          def _():
            prefetch(base + i + 1, nxt)

          wait_gathers(cur)
          compute_and_scatter(cur)
      return 0
    lax.fori_loop(0, nchunks, chunk, 0)
    wait_scatter(bufs[(nchunks - 1) & 1])
    plsc.subcore_barrier()

    pltpu.sync_copy(acc_sh.at[pl.ds(r0, rps)],
                    out_hbm.at[cid, pl.ds(r0, rps)])

  return agg


def kernel(x, edge_index, W1, att_src1, att_dst1, bias1,
           W2, att_src2, att_dst2, bias2):
  n, in_dim = x.shape
  e = edge_index.shape[1]
  heads, hid = att_src1.shape
  out_dim = W2.shape[1]
  f1 = heads * hid
  blk = 1000

  idx3d = jnp.stack([edge_index[0].reshape(-1, 100),
                     edge_index[1].reshape(-1, 100)], axis=1)

  head_of_col = jnp.arange(f1) // hid
  onehot = jnp.eye(heads, dtype=jnp.float32)[head_of_col]
  As = onehot * att_src1.reshape(f1)[:, None]
  Ad = onehot * att_dst1.reshape(f1)[:, None]
  pad8 = jnp.zeros((f1, 16 - heads), jnp.float32)
  As16 = jnp.concatenate([As, pad8], axis=1)
  Ad16 = jnp.concatenate([Ad, pad8], axis=1)
  Waug = jnp.concatenate([W1, jnp.dot(W1, As16)], axis=1)
  Wad = jnp.dot(W1, Ad16)
  expand = jnp.concatenate(
      [onehot.T, jnp.zeros((16 - heads, f1), jnp.float32)], axis=0)
  ones16 = jnp.ones((1, 16), jnp.float32)
  AS2rep = att_src2.reshape(out_dim, 1) @ ones16
  AD2rep = att_dst2.reshape(out_dim, 1) @ ones16
  W2aug = jnp.concatenate([jnp.eye(out_dim, dtype=jnp.float32), AS2rep],
                          axis=1)
  avg = jnp.full((16, out_dim), 1.0 / 16.0, jnp.float32)
  bias1r = bias1.reshape(1, f1)
  bias2r = bias2.reshape(1, out_dim)

  aug1, ad1 = _row_grid_call(
      _dense_a_body, n, blk,
      [(n, in_dim), (in_dim, f1 + 16), (in_dim, 16)],
      [(n, f1 + 16), (n, 16)],
      [True, False, False], [True, True])(x, Waug, Wad)

  accp1 = _make_edge_aggregator(n, e, f1, logit_replicated=False)(
      aug1, ad1, idx3d)

  aug2, ad2 = _row_grid_call(
      _dense_c_body, n, blk,
      [(2, n, f1 + 16), (16, f1), (1, f1), (f1, out_dim), (out_dim, out_dim + 16),
       (out_dim, 16)],
      [(n, out_dim + 16), (n, 16)],
      [True, False, False, False, False, False], [True, True])(
          accp1, expand, bias1r, W2, W2aug, AD2rep)

  accp2 = _make_edge_aggregator(n, e, out_dim, logit_replicated=True)(
      aug2, ad2, idx3d)

  out = _row_grid_call(
      _dense_e_body, n, blk,
      [(2, n, out_dim + 16), (16, out_dim), (1, out_dim)],
      [(n, out_dim)],
      [True, False, False], [True])(accp2, avg, bias2r)
  return out
